# baseline (speedup 1.0000x reference)
import functools

import jax
import jax.numpy as jnp
from jax import lax
from jax.experimental import pallas as pl
from jax.experimental.pallas import tpu as pltpu
from jax.experimental.pallas import tpu_sc as plsc

ALPHA = 0.1
_NC, _NS, _L = 2, 16, 16
_NW = _NC * _NS


def _sc_bitmap_body(idx_hbm, out_hbm, idx_kv, loc_v):
    KW, TW = loc_v.shape
    K = idx_kv.shape[0]
    W = KW // K
    wid = lax.axis_index("s") * _NC + lax.axis_index("c")
    base = wid * TW
    pltpu.sync_copy(idx_hbm.at[:, pl.ds(base, TW)], idx_kv)

    one = jnp.ones((_L,), jnp.int32)
    for k in range(K):
        for c in range(TW // _L):
            iv = idx_kv[k, pl.ds(c * _L, _L)]
            wv = iv >> 5
            bit = jnp.left_shift(one, iv & 31)
            for w in range(W):
                loc_v[k * W + w, pl.ds(c * _L, _L)] = jnp.where(
                    wv == w, bit, 0)

    pltpu.sync_copy(loc_v, out_hbm.at[:, pl.ds(base, TW)])


def _mb_kernel(bk_ref, hid_ref, mem_ref, out_ref, sums_ref, cnt_ref):
    i = pl.program_id(0)
    nsteps = pl.num_programs(0)
    KW, TB = bk_ref.shape
    N = mem_ref.shape[0]
    W = N // 32

    @pl.when(i == 0)
    def _init():
        sums_ref[...] = jnp.zeros_like(sums_ref)
        cnt_ref[...] = jnp.zeros_like(cnt_ref)

    bk = bk_ref[...]
    bits = bk[0:W, :]
    for k in range(1, KW // W):
        bits = bits | bk[k * W : (k + 1) * W, :]

    one_bits = jnp.int32(0x3F800000)
    sub_iota = jax.lax.broadcasted_iota(jnp.int32, (32, 1), 0)
    pattern = jnp.left_shift(jnp.int32(1), sub_iota)
    groups = []
    for g in range(W):
        hit = (bits[g : g + 1, :] & pattern) != 0
        groups.append(jnp.where(hit, one_bits, 0))
    maskf = jax.lax.bitcast_convert_type(
        jnp.concatenate(groups, axis=0), jnp.float32)

    cnt_ref[...] += jnp.sum(maskf, axis=1, keepdims=True)
    sums_ref[...] += jax.lax.dot_general(
        maskf, hid_ref[...], (((1,), (0,)), ((), ())),
        preferred_element_type=jnp.float32).astype(jnp.bfloat16)

    @pl.when(i == nsteps - 1)
    def _finish():
        counts = cnt_ref[...]
        agg = sums_ref[...].astype(jnp.float32) / jnp.maximum(counts, 1.0)
        memf = mem_ref[...].astype(jnp.float32)
        upd = ALPHA * agg + (1.0 - ALPHA) * memf
        out_ref[...] = jnp.where(counts > 0.0, upd, memf).astype(jnp.bfloat16)


def kernel(hidden_states, batch_idx, slot_indices, memory):
    T, D = hidden_states.shape
    K = slot_indices.shape[1]
    N = memory.shape[1]
    W = N // 32
    TW = T // _NW
    TB = 1024
    idx_t = slot_indices.T.astype(jnp.int32)
    mem2d = memory[0]

    sc_bitmap = functools.partial(
        pl.kernel,
        out_type=jax.ShapeDtypeStruct((K * W, T), jnp.int32),
        mesh=plsc.VectorSubcoreMesh(core_axis_name="c", subcore_axis_name="s"),
        scratch_types=[
            pltpu.VMEM((K, TW), jnp.int32),
            pltpu.VMEM((K * W, TW), jnp.int32),
        ],
    )(_sc_bitmap_body)
    bitmap = sc_bitmap(idx_t)

    out = pl.pallas_call(
        _mb_kernel,
        grid=(T // TB,),
        in_specs=[
            pl.BlockSpec((K * W, TB), lambda i: (0, i)),
            pl.BlockSpec((TB, D), lambda i: (i, 0)),
            pl.BlockSpec((N, D), lambda i: (0, 0)),
        ],
        out_specs=pl.BlockSpec((N, D), lambda i: (0, 0)),
        out_shape=jax.ShapeDtypeStruct((N, D), jnp.bfloat16),
        scratch_shapes=[
            pltpu.VMEM((N, D), jnp.bfloat16),
            pltpu.VMEM((N, 1), jnp.float32),
        ],
    )(bitmap, hidden_states, mem2d)
    return out[None]


# speedup vs baseline: 2.2474x; 2.2474x over previous
import jax
import jax.numpy as jnp
from jax.experimental import pallas as pl
from jax.experimental.pallas import tpu as pltpu

ALPHA = 0.1


def _mb_kernel(idx_ref, hid_ref, mem_ref, out_ref, bits_ref, sums_ref, cnt_ref):
    i = pl.program_id(0)
    nsteps = pl.num_programs(0)
    K, T = idx_ref.shape
    TB = hid_ref.shape[0]
    N = mem_ref.shape[0]
    W = N // 32

    @pl.when(i == 0)
    def _init():
        sums_ref[...] = jnp.zeros_like(sums_ref)
        cnt_ref[...] = jnp.zeros_like(cnt_ref)
        idx = idx_ref[...]
        w_iota = jax.lax.broadcasted_iota(jnp.int32, (W, T), 0)
        bits = jnp.zeros((W, T), jnp.int32)
        for k in range(K):
            col = idx[k : k + 1, :]
            bitk = jnp.left_shift(jnp.int32(1), col & 31)
            bits = bits | jnp.where((col >> 5) == w_iota, bitk, 0)
        bits_ref[...] = bits

    one_bits = jnp.int32(0x3F800000)
    bits = bits_ref[:, pl.ds(i * TB, TB)]
    sub_iota = jax.lax.broadcasted_iota(jnp.int32, (32, 1), 0)
    pattern = jnp.left_shift(jnp.int32(1), sub_iota)
    groups = []
    for g in range(W):
        hit = (bits[g : g + 1, :] & pattern) != 0
        groups.append(jnp.where(hit, one_bits, 0))
    maskf = jax.lax.bitcast_convert_type(
        jnp.concatenate(groups, axis=0), jnp.float32)

    cnt_ref[...] += jnp.sum(maskf, axis=1, keepdims=True)
    sums_ref[...] += jax.lax.dot_general(
        maskf, hid_ref[...], (((1,), (0,)), ((), ())),
        preferred_element_type=jnp.float32)

    @pl.when(i == nsteps - 1)
    def _finish():
        counts = cnt_ref[...]
        agg = sums_ref[...] / jnp.maximum(counts, 1.0)
        memf = mem_ref[...].astype(jnp.float32)
        upd = ALPHA * agg + (1.0 - ALPHA) * memf
        out_ref[...] = jnp.where(counts > 0.0, upd, memf).astype(jnp.bfloat16)


def kernel(hidden_states, batch_idx, slot_indices, memory):
    T, D = hidden_states.shape
    K = slot_indices.shape[1]
    N = memory.shape[1]
    TB = 1024
    idx_t = slot_indices.T.astype(jnp.int32)
    mem2d = memory[0]
    out = pl.pallas_call(
        _mb_kernel,
        grid=(T // TB,),
        in_specs=[
            pl.BlockSpec((K, T), lambda i: (0, 0)),
            pl.BlockSpec((TB, D), lambda i: (i, 0)),
            pl.BlockSpec((N, D), lambda i: (0, 0)),
        ],
        out_specs=pl.BlockSpec((N, D), lambda i: (0, 0)),
        out_shape=jax.ShapeDtypeStruct((N, D), jnp.bfloat16),
        scratch_shapes=[
            pltpu.VMEM((N // 32, T), jnp.int32),
            pltpu.VMEM((N, D), jnp.float32),
            pltpu.VMEM((N, 1), jnp.float32),
        ],
    )(idx_t, hidden_states, mem2d)
    return out[None]


# speedup vs baseline: 2.3019x; 1.0242x over previous
import jax
import jax.numpy as jnp
from jax.experimental import pallas as pl
from jax.experimental.pallas import tpu as pltpu

ALPHA = 0.1


def _mb_kernel(idx_ref, hid_ref, mem_ref, out_ref, sums_ref, cnt_ref):
    i = pl.program_id(0)
    nsteps = pl.num_programs(0)
    K, TB = idx_ref.shape
    N = mem_ref.shape[0]
    W = N // 32

    @pl.when(i == 0)
    def _init():
        sums_ref[...] = jnp.zeros_like(sums_ref)
        cnt_ref[...] = jnp.zeros_like(cnt_ref)

    idx = idx_ref[...]
    one_bits = jnp.int32(0x3F800000)

    w_iota = jax.lax.broadcasted_iota(jnp.int32, (W, TB), 0)
    bits = jnp.zeros((W, TB), jnp.int32)
    for k in range(K):
        col = idx[k : k + 1, :]
        bitk = jnp.left_shift(jnp.int32(1), col & 31)
        bits = bits | jnp.where((col >> 5) == w_iota, bitk, 0)

    sub_iota = jax.lax.broadcasted_iota(jnp.int32, (32, 1), 0)
    pattern = jnp.left_shift(jnp.int32(1), sub_iota)
    groups = []
    for g in range(W):
        hit = (bits[g : g + 1, :] & pattern) != 0
        groups.append(jnp.where(hit, one_bits, 0))
    maskf = jax.lax.bitcast_convert_type(
        jnp.concatenate(groups, axis=0), jnp.float32)

    cnt_ref[...] += jnp.sum(maskf, axis=1, keepdims=True)
    sums_ref[...] += jax.lax.dot_general(
        maskf, hid_ref[...], (((1,), (0,)), ((), ())),
        preferred_element_type=jnp.float32)

    @pl.when(i == nsteps - 1)
    def _finish():
        counts = cnt_ref[...]
        agg = sums_ref[...] / jnp.maximum(counts, 1.0)
        memf = mem_ref[...].astype(jnp.float32)
        upd = ALPHA * agg + (1.0 - ALPHA) * memf
        out_ref[...] = jnp.where(counts > 0.0, upd, memf).astype(jnp.bfloat16)


def kernel(hidden_states, batch_idx, slot_indices, memory):
    T, D = hidden_states.shape
    K = slot_indices.shape[1]
    N = memory.shape[1]
    TB = 1024
    idx_t = slot_indices.T.astype(jnp.int32)
    mem2d = memory[0]
    out = pl.pallas_call(
        _mb_kernel,
        grid=(T // TB,),
        in_specs=[
            pl.BlockSpec((K, TB), lambda i: (0, i)),
            pl.BlockSpec((TB, D), lambda i: (i, 0)),
            pl.BlockSpec((N, D), lambda i: (0, 0)),
        ],
        out_specs=pl.BlockSpec((N, D), lambda i: (0, 0)),
        out_shape=jax.ShapeDtypeStruct((N, D), jnp.bfloat16),
        scratch_shapes=[
            pltpu.VMEM((N, D), jnp.float32),
            pltpu.VMEM((N, 1), jnp.float32),
        ],
    )(idx_t, hidden_states, mem2d)
    return out[None]
